# speedup vs baseline: 1.9023x; 1.9023x over previous
import functools

import jax
import jax.numpy as jnp
from jax import lax
from jax.experimental import pallas as pl
from jax.experimental.pallas import tpu as pltpu
from jax.experimental.pallas import tpu_sc as plsc

_NC = 2
_NS = 16
_NW = _NC * _NS


def _sc_gather(user_ids, item_ids, user_emb, item_emb):
    B = user_ids.shape[0]
    D = user_emb.shape[1]
    b_per_w = B // _NW
    mesh = plsc.VectorSubcoreMesh(
        core_axis_name="c", subcore_axis_name="s", num_cores=_NC, num_subcores=_NS
    )

    @functools.partial(
        pl.kernel,
        mesh=mesh,
        out_type=(
            jax.ShapeDtypeStruct((B, D), jnp.float32),
            jax.ShapeDtypeStruct((B, D), jnp.float32),
        ),
        scratch_types=[
            pltpu.VMEM((b_per_w,), jnp.int32),
            pltpu.VMEM((b_per_w, D), jnp.float32),
            pltpu.SemaphoreType.DMA,
        ],
    )
    def gather_kernel(uid_hbm, iid_hbm, uemb_hbm, iemb_hbm, uout_hbm, iout_hbm,
                      idx_v, rows_v, sem):
        wid = lax.axis_index("s") * _NC + lax.axis_index("c")
        base = wid * b_per_w
        pltpu.sync_copy(uid_hbm.at[pl.ds(base, b_per_w)], idx_v)
        pltpu.async_copy(uemb_hbm.at[idx_v], rows_v, sem).wait()
        pltpu.sync_copy(rows_v, uout_hbm.at[pl.ds(base, b_per_w)])
        pltpu.sync_copy(iid_hbm.at[pl.ds(base, b_per_w)], idx_v)
        pltpu.async_copy(iemb_hbm.at[idx_v], rows_v, sem).wait()
        pltpu.sync_copy(rows_v, iout_hbm.at[pl.ds(base, b_per_w)])

    return gather_kernel(user_ids, item_ids, user_emb, item_emb)


def _mlp_body(u_ref, i_ref, w1u_ref, w1i_ref, b1_ref, w2_ref, b2_ref,
              w3_ref, b3_ref, w4_ref, b4_ref, out_ref):
    x = jnp.dot(u_ref[...], w1u_ref[...], preferred_element_type=jnp.float32)
    x = x + jnp.dot(i_ref[...], w1i_ref[...], preferred_element_type=jnp.float32)
    h = jnp.maximum(x + b1_ref[...], 0.0)
    enc = jnp.dot(h, w2_ref[...], preferred_element_type=jnp.float32) + b2_ref[...]
    h2 = jnp.maximum(
        jnp.dot(enc, w3_ref[...], preferred_element_type=jnp.float32) + b3_ref[...],
        0.0,
    )
    out_ref[...] = jnp.sum(h2 * w4_ref[...], axis=1, keepdims=True) + b4_ref[...]


def _mlp(u, i, W1u, W1i, b1, W2, b2, W3, b3, w4row, b4, tile_m=1024):
    B, D = u.shape
    H = W1u.shape[1]
    L = W2.shape[1]
    grid = (B // tile_m,)
    fixed = lambda m: (0, 0)
    out = pl.pallas_call(
        _mlp_body,
        grid=grid,
        in_specs=[
            pl.BlockSpec((tile_m, D), lambda m: (m, 0)),
            pl.BlockSpec((tile_m, D), lambda m: (m, 0)),
            pl.BlockSpec((D, H), fixed),
            pl.BlockSpec((D, H), fixed),
            pl.BlockSpec((1, H), fixed),
            pl.BlockSpec((H, L), fixed),
            pl.BlockSpec((1, L), fixed),
            pl.BlockSpec((L, H), fixed),
            pl.BlockSpec((1, H), fixed),
            pl.BlockSpec((1, H), fixed),
            pl.BlockSpec((1, 1), fixed),
        ],
        out_specs=pl.BlockSpec((tile_m, 1), lambda m: (m, 0)),
        out_shape=jax.ShapeDtypeStruct((B, 1), jnp.float32),
    )(u, i, W1u, W1i, b1, W2, b2, W3, b3, w4row, b4)
    return out


def kernel(user_ids, item_ids, user_emb, item_emb, W1, b1, W2, b2, W3, b3, W4, b4):
    B = user_ids.shape[0]
    D = user_emb.shape[1]
    u, i = _sc_gather(user_ids, item_ids, user_emb, item_emb)
    W1u = W1[:D]
    W1i = W1[D:]
    out = _mlp(
        u, i, W1u, W1i,
        b1.reshape(1, -1), W2, b2.reshape(1, -1), W3, b3.reshape(1, -1),
        W4.reshape(1, -1), b4.reshape(1, 1),
    )
    return jnp.reshape(out, (B,))


# speedup vs baseline: 1.9098x; 1.0040x over previous
import functools

import jax
import jax.numpy as jnp
from jax import lax
from jax.experimental import pallas as pl
from jax.experimental.pallas import tpu as pltpu
from jax.experimental.pallas import tpu_sc as plsc

_NC = 2
_NS = 16
_NW = _NC * _NS


def _sc_gather(user_ids, item_ids, user_emb, item_emb):
    B = user_ids.shape[0]
    D = user_emb.shape[1]
    b_per_w = B // _NW
    mesh = plsc.VectorSubcoreMesh(
        core_axis_name="c", subcore_axis_name="s", num_cores=_NC, num_subcores=_NS
    )

    @functools.partial(
        pl.kernel,
        mesh=mesh,
        out_type=(
            jax.ShapeDtypeStruct((B, D), jnp.float32),
            jax.ShapeDtypeStruct((B, D), jnp.float32),
        ),
        scratch_types=[
            pltpu.VMEM((b_per_w,), jnp.int32),
            pltpu.VMEM((b_per_w, D), jnp.float32),
            pltpu.SemaphoreType.DMA,
        ],
    )
    def gather_kernel(uid_hbm, iid_hbm, uemb_hbm, iemb_hbm, uout_hbm, iout_hbm,
                      idx_v, rows_v, sem):
        wid = lax.axis_index("s") * _NC + lax.axis_index("c")
        base = wid * b_per_w
        pltpu.sync_copy(uid_hbm.at[pl.ds(base, b_per_w)], idx_v)
        pltpu.async_copy(uemb_hbm.at[idx_v], rows_v, sem).wait()
        pltpu.sync_copy(rows_v, uout_hbm.at[pl.ds(base, b_per_w)])
        pltpu.sync_copy(iid_hbm.at[pl.ds(base, b_per_w)], idx_v)
        pltpu.async_copy(iemb_hbm.at[idx_v], rows_v, sem).wait()
        pltpu.sync_copy(rows_v, iout_hbm.at[pl.ds(base, b_per_w)])

    return gather_kernel(user_ids, item_ids, user_emb, item_emb)


def _mlp_body(u_ref, i_ref, w1u_ref, w1i_ref, b1_ref, w2_ref, b2_ref,
              w3_ref, b3_ref, w4_ref, b4_ref, out_ref):
    bf = jnp.bfloat16
    x = jnp.dot(u_ref[...].astype(bf), w1u_ref[...].astype(bf),
                preferred_element_type=jnp.float32)
    x = x + jnp.dot(i_ref[...].astype(bf), w1i_ref[...].astype(bf),
                    preferred_element_type=jnp.float32)
    h = jnp.maximum(x + b1_ref[...], 0.0)
    enc = jnp.dot(h.astype(bf), w2_ref[...].astype(bf),
                  preferred_element_type=jnp.float32) + b2_ref[...]
    h2 = jnp.maximum(
        jnp.dot(enc.astype(bf), w3_ref[...].astype(bf),
                preferred_element_type=jnp.float32) + b3_ref[...],
        0.0,
    )
    out_ref[...] = jnp.sum(h2 * w4_ref[...], axis=1, keepdims=True) + b4_ref[...]


def _mlp(u, i, W1u, W1i, b1, W2, b2, W3, b3, w4row, b4, tile_m=1024):
    B, D = u.shape
    H = W1u.shape[1]
    L = W2.shape[1]
    grid = (B // tile_m,)
    fixed = lambda m: (0, 0)
    out = pl.pallas_call(
        _mlp_body,
        grid=grid,
        in_specs=[
            pl.BlockSpec((tile_m, D), lambda m: (m, 0)),
            pl.BlockSpec((tile_m, D), lambda m: (m, 0)),
            pl.BlockSpec((D, H), fixed),
            pl.BlockSpec((D, H), fixed),
            pl.BlockSpec((1, H), fixed),
            pl.BlockSpec((H, L), fixed),
            pl.BlockSpec((1, L), fixed),
            pl.BlockSpec((L, H), fixed),
            pl.BlockSpec((1, H), fixed),
            pl.BlockSpec((1, H), fixed),
            pl.BlockSpec((1, 1), fixed),
        ],
        out_specs=pl.BlockSpec((tile_m, 1), lambda m: (m, 0)),
        out_shape=jax.ShapeDtypeStruct((B, 1), jnp.float32),
    )(u, i, W1u, W1i, b1, W2, b2, W3, b3, w4row, b4)
    return out


def kernel(user_ids, item_ids, user_emb, item_emb, W1, b1, W2, b2, W3, b3, W4, b4):
    B = user_ids.shape[0]
    D = user_emb.shape[1]
    u, i = _sc_gather(user_ids, item_ids, user_emb, item_emb)
    W1u = W1[:D]
    W1i = W1[D:]
    out = _mlp(
        u, i, W1u, W1i,
        b1.reshape(1, -1), W2, b2.reshape(1, -1), W3, b3.reshape(1, -1),
        W4.reshape(1, -1), b4.reshape(1, 1),
    )
    return jnp.reshape(out, (B,))


# speedup vs baseline: 1.9851x; 1.0394x over previous
import functools

import jax
import jax.numpy as jnp
from jax import lax
from jax.experimental import pallas as pl
from jax.experimental.pallas import tpu as pltpu
from jax.experimental.pallas import tpu_sc as plsc

_NC = 2
_NS = 16
_NW = _NC * _NS


def _sc_gather(user_ids, item_ids, user_emb, item_emb):
    B = user_ids.shape[0]
    D = user_emb.shape[1]
    b_per_w = B // _NW
    mesh = plsc.VectorSubcoreMesh(
        core_axis_name="c", subcore_axis_name="s", num_cores=_NC, num_subcores=_NS
    )

    @functools.partial(
        pl.kernel,
        mesh=mesh,
        out_type=(
            jax.ShapeDtypeStruct((B, D), jnp.float32),
            jax.ShapeDtypeStruct((B, D), jnp.float32),
        ),
        scratch_types=[
            pltpu.VMEM((b_per_w,), jnp.int32),
            pltpu.VMEM((b_per_w, D), jnp.float32),
            pltpu.SemaphoreType.DMA,
        ],
    )
    def gather_kernel(uid_hbm, iid_hbm, uemb_hbm, iemb_hbm, uout_hbm, iout_hbm,
                      idx_v, rows_v, sem):
        wid = lax.axis_index("s") * _NC + lax.axis_index("c")
        base = wid * b_per_w
        pltpu.sync_copy(uid_hbm.at[pl.ds(base, b_per_w)], idx_v)
        pltpu.async_copy(uemb_hbm.at[idx_v], rows_v, sem).wait()
        pltpu.sync_copy(rows_v, uout_hbm.at[pl.ds(base, b_per_w)])
        pltpu.sync_copy(iid_hbm.at[pl.ds(base, b_per_w)], idx_v)
        pltpu.async_copy(iemb_hbm.at[idx_v], rows_v, sem).wait()
        pltpu.sync_copy(rows_v, iout_hbm.at[pl.ds(base, b_per_w)])

    return gather_kernel(user_ids, item_ids, user_emb, item_emb)


def _mlp_body(u_ref, i_ref, w1u_ref, w1i_ref, b1_ref, w2_ref, b2_ref,
              w3_ref, b3_ref, w4_ref, b4_ref, out_ref):
    bf = jnp.bfloat16
    x = jnp.dot(u_ref[...].astype(bf), w1u_ref[...].astype(bf),
                preferred_element_type=jnp.float32)
    x = x + jnp.dot(i_ref[...].astype(bf), w1i_ref[...].astype(bf),
                    preferred_element_type=jnp.float32)
    h = jnp.maximum(x + b1_ref[...], 0.0)
    enc = jnp.dot(h.astype(bf), w2_ref[...].astype(bf),
                  preferred_element_type=jnp.float32) + b2_ref[...]
    h2 = jnp.maximum(
        jnp.dot(enc.astype(bf), w3_ref[...].astype(bf),
                preferred_element_type=jnp.float32) + b3_ref[...],
        0.0,
    )
    out_ref[...] = jnp.sum(h2 * w4_ref[...], axis=1, keepdims=True) + b4_ref[...]


def _mlp(u, i, W1u, W1i, b1, W2, b2, W3, b3, w4row, b4, tile_m=2048):
    B, D = u.shape
    H = W1u.shape[1]
    L = W2.shape[1]
    grid = (B // tile_m,)
    fixed = lambda m: (0, 0)
    out = pl.pallas_call(
        _mlp_body,
        grid=grid,
        in_specs=[
            pl.BlockSpec((tile_m, D), lambda m: (m, 0)),
            pl.BlockSpec((tile_m, D), lambda m: (m, 0)),
            pl.BlockSpec((D, H), fixed),
            pl.BlockSpec((D, H), fixed),
            pl.BlockSpec((1, H), fixed),
            pl.BlockSpec((H, L), fixed),
            pl.BlockSpec((1, L), fixed),
            pl.BlockSpec((L, H), fixed),
            pl.BlockSpec((1, H), fixed),
            pl.BlockSpec((1, H), fixed),
            pl.BlockSpec((1, 1), fixed),
        ],
        out_specs=pl.BlockSpec((tile_m, 1), lambda m: (m, 0)),
        out_shape=jax.ShapeDtypeStruct((B, 1), jnp.float32),
    )(u, i, W1u, W1i, b1, W2, b2, W3, b3, w4row, b4)
    return out


def kernel(user_ids, item_ids, user_emb, item_emb, W1, b1, W2, b2, W3, b3, W4, b4):
    B = user_ids.shape[0]
    D = user_emb.shape[1]
    u, i = _sc_gather(user_ids, item_ids, user_emb, item_emb)
    W1u = W1[:D]
    W1i = W1[D:]
    out = _mlp(
        u, i, W1u, W1i,
        b1.reshape(1, -1), W2, b2.reshape(1, -1), W3, b3.reshape(1, -1),
        W4.reshape(1, -1), b4.reshape(1, 1),
    )
    return jnp.reshape(out, (B,))


# speedup vs baseline: 2.0124x; 1.0138x over previous
import functools

import jax
import jax.numpy as jnp
from jax import lax
from jax.experimental import pallas as pl
from jax.experimental.pallas import tpu as pltpu
from jax.experimental.pallas import tpu_sc as plsc

_NC = 2
_NS = 16
_NW = _NC * _NS


def _sc_gather(user_ids, item_ids, user_emb, item_emb):
    B = user_ids.shape[0]
    D = user_emb.shape[1]
    b_per_w = B // _NW
    mesh = plsc.VectorSubcoreMesh(
        core_axis_name="c", subcore_axis_name="s", num_cores=_NC, num_subcores=_NS
    )

    @functools.partial(
        pl.kernel,
        mesh=mesh,
        out_type=(
            jax.ShapeDtypeStruct((B, D), jnp.float32),
            jax.ShapeDtypeStruct((B, D), jnp.float32),
        ),
        scratch_types=[
            pltpu.VMEM((b_per_w,), jnp.int32),
            pltpu.VMEM((b_per_w, D), jnp.float32),
            pltpu.SemaphoreType.DMA,
        ],
    )
    def gather_kernel(uid_hbm, iid_hbm, uemb_hbm, iemb_hbm, uout_hbm, iout_hbm,
                      idx_v, rows_v, sem):
        wid = lax.axis_index("s") * _NC + lax.axis_index("c")
        base = wid * b_per_w
        pltpu.sync_copy(uid_hbm.at[pl.ds(base, b_per_w)], idx_v)
        pltpu.async_copy(uemb_hbm.at[idx_v], rows_v, sem).wait()
        pltpu.sync_copy(rows_v, uout_hbm.at[pl.ds(base, b_per_w)])
        pltpu.sync_copy(iid_hbm.at[pl.ds(base, b_per_w)], idx_v)
        pltpu.async_copy(iemb_hbm.at[idx_v], rows_v, sem).wait()
        pltpu.sync_copy(rows_v, iout_hbm.at[pl.ds(base, b_per_w)])

    return gather_kernel(user_ids, item_ids, user_emb, item_emb)


def _mlp_body(u_ref, i_ref, w1u_ref, w1i_ref, b1_ref, w2_ref, b2_ref,
              w3_ref, b3_ref, w4_ref, b4_ref, out_ref):
    bf = jnp.bfloat16
    x = jnp.dot(u_ref[...].astype(bf), w1u_ref[...].astype(bf),
                preferred_element_type=jnp.float32)
    x = x + jnp.dot(i_ref[...].astype(bf), w1i_ref[...].astype(bf),
                    preferred_element_type=jnp.float32)
    h = jnp.maximum(x + b1_ref[...], 0.0)
    enc = jnp.dot(h.astype(bf), w2_ref[...].astype(bf),
                  preferred_element_type=jnp.float32) + b2_ref[...]
    h2 = jnp.maximum(
        jnp.dot(enc.astype(bf), w3_ref[...].astype(bf),
                preferred_element_type=jnp.float32) + b3_ref[...],
        0.0,
    )
    out_ref[...] = jnp.sum(h2 * w4_ref[...], axis=1, keepdims=True) + b4_ref[...]


def _mlp(u, i, W1u, W1i, b1, W2, b2, W3, b3, w4row, b4, tile_m=4096):
    B, D = u.shape
    H = W1u.shape[1]
    L = W2.shape[1]
    grid = (B // tile_m,)
    fixed = lambda m: (0, 0)
    out = pl.pallas_call(
        _mlp_body,
        grid=grid,
        in_specs=[
            pl.BlockSpec((tile_m, D), lambda m: (m, 0)),
            pl.BlockSpec((tile_m, D), lambda m: (m, 0)),
            pl.BlockSpec((D, H), fixed),
            pl.BlockSpec((D, H), fixed),
            pl.BlockSpec((1, H), fixed),
            pl.BlockSpec((H, L), fixed),
            pl.BlockSpec((1, L), fixed),
            pl.BlockSpec((L, H), fixed),
            pl.BlockSpec((1, H), fixed),
            pl.BlockSpec((1, H), fixed),
            pl.BlockSpec((1, 1), fixed),
        ],
        out_specs=pl.BlockSpec((tile_m, 1), lambda m: (m, 0)),
        out_shape=jax.ShapeDtypeStruct((B, 1), jnp.float32),
    )(u, i, W1u, W1i, b1, W2, b2, W3, b3, w4row, b4)
    return out


def kernel(user_ids, item_ids, user_emb, item_emb, W1, b1, W2, b2, W3, b3, W4, b4):
    B = user_ids.shape[0]
    D = user_emb.shape[1]
    u, i = _sc_gather(user_ids, item_ids, user_emb, item_emb)
    W1u = W1[:D]
    W1i = W1[D:]
    out = _mlp(
        u, i, W1u, W1i,
        b1.reshape(1, -1), W2, b2.reshape(1, -1), W3, b3.reshape(1, -1),
        W4.reshape(1, -1), b4.reshape(1, 1),
    )
    return jnp.reshape(out, (B,))


# speedup vs baseline: 2.2140x; 1.1002x over previous
import functools

import jax
import jax.numpy as jnp
from jax import lax
from jax.experimental import pallas as pl
from jax.experimental.pallas import tpu as pltpu
from jax.experimental.pallas import tpu_sc as plsc

_NC = 2
_NS = 16
_NW = _NC * _NS


def _sc_gather(user_ids, item_ids, user_emb, item_emb):
    B = user_ids.shape[0]
    D = user_emb.shape[1]
    b_per_w = B // _NW
    mesh = plsc.VectorSubcoreMesh(
        core_axis_name="c", subcore_axis_name="s", num_cores=_NC, num_subcores=_NS
    )

    @functools.partial(
        pl.kernel,
        mesh=mesh,
        out_type=jax.ShapeDtypeStruct((B, 2 * D), jnp.float32),
        scratch_types=[
            pltpu.VMEM((b_per_w,), jnp.int32),
            pltpu.VMEM((b_per_w, D), jnp.float32),
            pltpu.SemaphoreType.DMA,
        ],
    )
    def gather_kernel(uid_hbm, iid_hbm, uemb_hbm, iemb_hbm, out_hbm,
                      idx_v, rows_v, sem):
        wid = lax.axis_index("s") * _NC + lax.axis_index("c")
        base = wid * b_per_w
        pltpu.sync_copy(uid_hbm.at[pl.ds(base, b_per_w)], idx_v)
        pltpu.async_copy(uemb_hbm.at[idx_v], rows_v, sem).wait()
        pltpu.sync_copy(rows_v, out_hbm.at[pl.ds(base, b_per_w), pl.ds(0, D)])
        pltpu.sync_copy(iid_hbm.at[pl.ds(base, b_per_w)], idx_v)
        pltpu.async_copy(iemb_hbm.at[idx_v], rows_v, sem).wait()
        pltpu.sync_copy(rows_v, out_hbm.at[pl.ds(base, b_per_w), pl.ds(D, D)])

    return gather_kernel(user_ids, item_ids, user_emb, item_emb)


def _mlp_body(x_ref, w1_ref, b1_ref, w2_ref, b2_ref,
              w3_ref, b3_ref, w4_ref, b4_ref, out_ref):
    bf = jnp.bfloat16
    x = jnp.dot(x_ref[...].astype(bf), w1_ref[...].astype(bf),
                preferred_element_type=jnp.float32)
    h = jnp.maximum(x + b1_ref[...], 0.0)
    enc = jnp.dot(h.astype(bf), w2_ref[...].astype(bf),
                  preferred_element_type=jnp.float32) + b2_ref[...]
    h2 = jnp.maximum(
        jnp.dot(enc.astype(bf), w3_ref[...].astype(bf),
                preferred_element_type=jnp.float32) + b3_ref[...],
        0.0,
    )
    out_ref[...] = jnp.sum(h2 * w4_ref[...], axis=1, keepdims=True) + b4_ref[...]


def _mlp(x, W1, b1, W2, b2, W3, b3, w4row, b4, tile_m=4096):
    B, D2 = x.shape
    H = W1.shape[1]
    L = W2.shape[1]
    grid = (B // tile_m,)
    fixed = lambda m: (0, 0)
    out = pl.pallas_call(
        _mlp_body,
        grid=grid,
        in_specs=[
            pl.BlockSpec((tile_m, D2), lambda m: (m, 0)),
            pl.BlockSpec((D2, H), fixed),
            pl.BlockSpec((1, H), fixed),
            pl.BlockSpec((H, L), fixed),
            pl.BlockSpec((1, L), fixed),
            pl.BlockSpec((L, H), fixed),
            pl.BlockSpec((1, H), fixed),
            pl.BlockSpec((1, H), fixed),
            pl.BlockSpec((1, 1), fixed),
        ],
        out_specs=pl.BlockSpec((tile_m, 1), lambda m: (m, 0)),
        out_shape=jax.ShapeDtypeStruct((B, 1), jnp.float32),
    )(x, W1, b1, W2, b2, W3, b3, w4row, b4)
    return out


def kernel(user_ids, item_ids, user_emb, item_emb, W1, b1, W2, b2, W3, b3, W4, b4):
    B = user_ids.shape[0]
    x = _sc_gather(user_ids, item_ids, user_emb, item_emb)
    out = _mlp(
        x, W1,
        b1.reshape(1, -1), W2, b2.reshape(1, -1), W3, b3.reshape(1, -1),
        W4.reshape(1, -1), b4.reshape(1, 1),
    )
    return jnp.reshape(out, (B,))
